# baseline (speedup 1.0000x reference)
import functools
import numpy as np
import jax
import jax.numpy as jnp
from jax import lax
from jax.experimental import pallas as pl
from jax.experimental.pallas import tpu as pltpu
from jax.experimental.pallas import tpu_sc as plsc

_ZS = (1, 6, 7, 8)
_CUTOFF = 5.5
_N = 10000
_E = 160000

_NT = 10240
_EPAD = 163840
_BSZ = 1024
_NBLK = _E // _BSZ
_ETAIL = _E - _NBLK * _BSZ
_ROWS_PER_SUB = _NT // 16
_PM = 30
_SCW = 96


def _build_consts():
    r1 = np.zeros((256, 540), np.float32)
    r2 = np.zeros((8, 540), np.float32)
    for j in range(6):
        for k in range(10):
            m = k * 6 + j
            part, mm = divmod(m, _PM)
            for a in range(3):
                scol = part * 128 + (mm // 2) * 8 + a * 2 + (mm % 2)
                for b in range(3):
                    c = (j * 10 + k) * 9 + a * 3 + b
                    r1[scol, c] = 1.0
                    r2[b, c] = 1.0
    rb = np.zeros((1080, 162), np.float32)
    l2_pref = {4: 1.0, 5: 2.0, 6: 2.0, 7: 1.0, 8: 2.0, 9: 1.0}
    for j in range(6):
        for c9 in range(9):
            rb[(j * 10 + 0) * 9 + c9, j * 27 + 0 * 9 + c9] = 1.0
            for k in (1, 2, 3):
                rb[540 + (j * 10 + k) * 9 + c9, j * 27 + 1 * 9 + c9] = 1.0
            for k, pref in l2_pref.items():
                rb[540 + (j * 10 + k) * 9 + c9, j * 27 + 2 * 9 + c9] = pref
    return r1, r2, rb


_R1, _R2, _RB = _build_consts()


def _embed_body(an_ref, w_ref, out_ref):
    an0 = an_ref[0:1, :]
    rows = []
    for a in range(3):
        acc = jnp.zeros(an0.shape, jnp.float32)
        for zi, z in enumerate(_ZS):
            acc = acc + jnp.where(an0 == z, w_ref[zi, a], 0.0)
        rows.append(acc)
    rows.append(jnp.zeros((5, an0.shape[1]), jnp.float32))
    out_ref[...] = jnp.concatenate(rows, axis=0)


def _embed(an8, w):
    return pl.pallas_call(
        _embed_body,
        out_shape=jax.ShapeDtypeStruct((8, _NT), jnp.float32),
        in_specs=[
            pl.BlockSpec(memory_space=pltpu.VMEM),
            pl.BlockSpec(memory_space=pltpu.SMEM),
        ],
        out_specs=pl.BlockSpec(memory_space=pltpu.VMEM),
    )(an8, w)


_BE = 512


def _edge_body(l_ref, v_ref, out_ref):
    r = jnp.transpose(l_ref[...])
    vt = jnp.transpose(v_ref[...])
    vx = vt[0:1, :]
    vy = vt[1:2, :]
    vz = vt[2:3, :]
    theta = (np.pi / _CUTOFF) * r
    s1 = jnp.sin(theta)
    c1 = jnp.cos(theta)
    sins = [s1, 2.0 * c1 * s1]
    for _ in range(4):
        sins.append(2.0 * c1 * sins[-1] - sins[-2])
    x = r * (1.0 / _CUTOFF)
    x2 = x * x
    x3 = x2 * x
    x6 = x3 * x3
    fc = 1.0 - 28.0 * x6 + 48.0 * x6 * x - 21.0 * x6 * x2
    fc = jnp.where(x < 1.0, fc, 0.0)
    pref = np.float32(np.sqrt(2.0 / _CUTOFF)) * fc / r
    rad = [pref * s for s in sins]
    one = jnp.ones(r.shape, jnp.float32)
    ang = [one, vx, vy, vz, vx * vx, vx * vy, vx * vz, vy * vy, vy * vz,
           vz * vz]
    zrow = jnp.zeros((1, r.shape[1]), jnp.float32)
    planes = []
    for r2 in range(32):
        cid, sid = divmod(r2, 16)
        pair = []
        for q in range(2):
            m = cid * _PM + 2 * sid + q
            pair.append(rad[m % 6] * ang[m // 6] if m % _PM == 2 * sid + q
                        and 2 * sid + q < _PM else zrow)
        planes.append(jnp.concatenate(pair, axis=0)[None])
    out_ref[...] = jnp.concatenate(planes, axis=0)


def _edgefeat(lengths, vectors):
    return pl.pallas_call(
        _edge_body,
        grid=((_E + _BE - 1) // _BE,),
        out_shape=jax.ShapeDtypeStruct((32, 2, _E), jnp.float32),
        in_specs=[pl.BlockSpec((_BE, 1), lambda i: (i, 0)),
                  pl.BlockSpec((_BE, 3), lambda i: (i, 0))],
        out_specs=pl.BlockSpec((32, 2, _BE), lambda i: (0, 0, i)),
    )(lengths, vectors)


def _sc_body(pt_hbm, ei_hbm, emb_hbm, out_hbm,
             srcA, dstA, pA0, pA1, srcB, dstB, pB0, pB1, acc,
             embx, emby, embz, semA, semB):
    cid = lax.axis_index("c")
    sid = lax.axis_index("s")
    r2 = cid * 16 + sid
    pbase = r2 * 2 * _E

    pltpu.sync_copy(emb_hbm.at[pl.ds(0, _NT)], embx)
    pltpu.sync_copy(emb_hbm.at[pl.ds(_NT, _NT)], emby)
    pltpu.sync_copy(emb_hbm.at[pl.ds(2 * _NT, _NT)], embz)

    zero16 = jnp.zeros((16,), jnp.float32)
    lane = lax.iota(jnp.int32, 16)

    def _zrow(i, c):
        plsc.store_scatter(acc, [i * 16 + lane], zero16)
        return c

    lax.fori_loop(0, (_NT * 8) // 16, _zrow, 0)

    def _fire(b, sbuf, dbuf, p0, p1, sem):
        eb = jnp.minimum(b, _NBLK - 1) * _BSZ
        pltpu.async_copy(ei_hbm.at[pl.ds(eb, _BSZ)], sbuf, sem)
        pltpu.async_copy(ei_hbm.at[pl.ds(_E + eb, _BSZ)], dbuf, sem)
        pltpu.async_copy(pt_hbm.at[pl.ds(pbase + eb, _BSZ)], p0, sem)
        pltpu.async_copy(pt_hbm.at[pl.ds(pbase + _E + eb, _BSZ)], p1, sem)

    def _wait(sbuf, dbuf, p0, p1, sem):
        pltpu.make_async_copy(ei_hbm.at[pl.ds(0, _BSZ)], sbuf, sem).wait()
        pltpu.make_async_copy(ei_hbm.at[pl.ds(0, _BSZ)], dbuf, sem).wait()
        pltpu.make_async_copy(pt_hbm.at[pl.ds(0, _BSZ)], p0, sem).wait()
        pltpu.make_async_copy(pt_hbm.at[pl.ds(0, _BSZ)], p1, sem).wait()

    def _compute(sbuf, dbuf, p0, p1, ngroups=_BSZ // 16):
        for g in range(ngroups):
            sl = pl.ds(g * 16, 16)
            s16 = sbuf[sl]
            d16 = dbuf[sl]
            ex = plsc.load_gather(embx, [s16])
            ey = plsc.load_gather(emby, [s16])
            ez = plsc.load_gather(embz, [s16])
            d8 = d16 * 8
            for q, pm in ((0, p0[sl]), (1, p1[sl])):
                for a, ev in ((0, ex), (1, ey), (2, ez)):
                    plsc.addupdate_scatter(acc, [d8 + (a * 2 + q)], pm * ev)

    _fire(0, srcA, dstA, pA0, pA1, semA)

    def _body2(i, c):
        b0 = i * 2
        _fire(b0 + 1, srcB, dstB, pB0, pB1, semB)
        _wait(srcA, dstA, pA0, pA1, semA)
        _compute(srcA, dstA, pA0, pA1)
        _fire(b0 + 2, srcA, dstA, pA0, pA1, semA)
        _wait(srcB, dstB, pB0, pB1, semB)
        _compute(srcB, dstB, pB0, pB1)
        return c

    lax.fori_loop(0, _NBLK // 2, _body2, 0)
    _wait(srcA, dstA, pA0, pA1, semA)

    tb = _NBLK * _BSZ
    pltpu.sync_copy(ei_hbm.at[pl.ds(tb, _ETAIL)], srcA.at[pl.ds(0, _ETAIL)])
    pltpu.sync_copy(ei_hbm.at[pl.ds(_E + tb, _ETAIL)],
                    dstA.at[pl.ds(0, _ETAIL)])
    pltpu.sync_copy(pt_hbm.at[pl.ds(pbase + tb, _ETAIL)],
                    pA0.at[pl.ds(0, _ETAIL)])
    pltpu.sync_copy(pt_hbm.at[pl.ds(pbase + _E + tb, _ETAIL)],
                    pA1.at[pl.ds(0, _ETAIL)])
    _compute(srcA, dstA, pA0, pA1, ngroups=_ETAIL // 16)

    pltpu.sync_copy(acc, out_hbm.at[pl.ds(r2 * (_NT * 8), _NT * 8)])


_sc_call = functools.partial(
    pl.kernel,
    out_type=pltpu.HBM((32 * _NT * 8,), jnp.float32),
    mesh=plsc.VectorSubcoreMesh(core_axis_name="c", subcore_axis_name="s",
                                num_cores=2, num_subcores=16),
    compiler_params=pltpu.CompilerParams(needs_layout_passes=False),
    scratch_types=[
        pltpu.VMEM((_BSZ,), jnp.int32),
        pltpu.VMEM((_BSZ,), jnp.int32),
        pltpu.VMEM((_BSZ,), jnp.float32),
        pltpu.VMEM((_BSZ,), jnp.float32),
        pltpu.VMEM((_BSZ,), jnp.int32),
        pltpu.VMEM((_BSZ,), jnp.int32),
        pltpu.VMEM((_BSZ,), jnp.float32),
        pltpu.VMEM((_BSZ,), jnp.float32),
        pltpu.VMEM((_NT * 8,), jnp.float32),
        pltpu.VMEM((_NT,), jnp.float32),
        pltpu.VMEM((_NT,), jnp.float32),
        pltpu.VMEM((_NT,), jnp.float32),
        pltpu.SemaphoreType.DMA,
        pltpu.SemaphoreType.DMA,
    ],
)(_sc_body)


_BN = 256


def _combine_body(*refs):
    planes = refs[:32]
    emb_ref, r1_ref, r2_ref, rb_ref, a_ref, b_ref = refs[32:]
    s = jnp.concatenate([p[0, 0] for p in planes], axis=1)
    f = lax.dot_general(emb_ref[...], r2_ref[...], (((0,), (0,)), ((), ())),
                        preferred_element_type=jnp.float32)
    a = jnp.dot(s, r1_ref[...], preferred_element_type=jnp.float32) * f
    a_ref[...] = a
    aa = jnp.concatenate([a, a * a], axis=1)
    b_ref[...] = jnp.dot(aa, rb_ref[...], preferred_element_type=jnp.float32)


def _combine(s2, emb_cols, r1, r2, rb):
    plane_specs = [
        pl.BlockSpec((1, 1, _BN, 8), lambda i, c=c, s=s: (c, s, i, 0))
        for c in range(2) for s in range(16)
    ]
    return pl.pallas_call(
        _combine_body,
        grid=(_NT // _BN,),
        out_shape=(
            jax.ShapeDtypeStruct((_N, 540), jnp.float32),
            jax.ShapeDtypeStruct((_N, 162), jnp.float32),
        ),
        in_specs=plane_specs + [
            pl.BlockSpec((8, _BN), lambda i: (0, i)),
            pl.BlockSpec((256, 540), lambda i: (0, 0)),
            pl.BlockSpec((8, 540), lambda i: (0, 0)),
            pl.BlockSpec((1080, 162), lambda i: (0, 0)),
        ],
        out_specs=(
            pl.BlockSpec((_BN, 540), lambda i: (i, 0)),
            pl.BlockSpec((_BN, 162), lambda i: (i, 0)),
        ),
    )(*([s2] * 32), emb_cols, r1, r2, rb)


def kernel(positions, atomic_numbers, edge_index, edge_lengths, edge_vectors,
           W):
    an8 = jnp.pad(atomic_numbers.astype(jnp.int32).reshape(1, _N),
                  ((0, 7), (0, _NT - _N)))

    emb_cols = _embed(an8, W.astype(jnp.float32))
    pt = _edgefeat(edge_lengths.astype(jnp.float32),
                   edge_vectors.astype(jnp.float32))
    s2 = _sc_call(pt.reshape(-1), edge_index.astype(jnp.int32).reshape(-1),
                  emb_cols[0:3].reshape(-1)).reshape(2, 16, _NT, 8)
    node_a, node_b = _combine(s2, emb_cols,
                              jnp.asarray(_R1), jnp.asarray(_R2),
                              jnp.asarray(_RB))
    return node_a.reshape(_N, 6, 10, 9), node_b.reshape(_N, 6, 3, 9)


# speedup vs baseline: 1.1540x; 1.1540x over previous
import functools
import numpy as np
import jax
import jax.numpy as jnp
from jax import lax
from jax.experimental import pallas as pl
from jax.experimental.pallas import tpu as pltpu
from jax.experimental.pallas import tpu_sc as plsc

_ZS = (1, 6, 7, 8)
_CUTOFF = 5.5
_N = 10000
_E = 160000

_NT = 10240
_EPAD = 163840
_BSZ = 1024
_NBLK = _EPAD // _BSZ
_PM = 30


def _build_consts():
    r1 = np.zeros((256, 540), np.float32)
    r2 = np.zeros((8, 540), np.float32)
    for j in range(6):
        for k in range(10):
            m = k * 6 + j
            part, mm = divmod(m, _PM)
            for a in range(3):
                scol = part * 128 + (mm // 2) * 8 + a * 2 + (mm % 2)
                for b in range(3):
                    c = (j * 10 + k) * 9 + a * 3 + b
                    r1[scol, c] = 1.0
                    r2[b, c] = 1.0
    rb = np.zeros((1080, 162), np.float32)
    l2_pref = {4: 1.0, 5: 2.0, 6: 2.0, 7: 1.0, 8: 2.0, 9: 1.0}
    for j in range(6):
        for c9 in range(9):
            rb[(j * 10 + 0) * 9 + c9, j * 27 + 0 * 9 + c9] = 1.0
            for k in (1, 2, 3):
                rb[540 + (j * 10 + k) * 9 + c9, j * 27 + 1 * 9 + c9] = 1.0
            for k, pref in l2_pref.items():
                rb[540 + (j * 10 + k) * 9 + c9, j * 27 + 2 * 9 + c9] = pref
    return r1, r2, rb


_R1, _R2, _RB = _build_consts()


def _embed_body(an_ref, w_ref, out_ref):
    an0 = an_ref[0:1, :]
    rows = []
    for a in range(3):
        acc = jnp.zeros(an0.shape, jnp.float32)
        for zi, z in enumerate(_ZS):
            acc = acc + jnp.where(an0 == z, w_ref[zi, a], 0.0)
        rows.append(acc)
    rows.append(jnp.zeros((5, an0.shape[1]), jnp.float32))
    out_ref[...] = jnp.concatenate(rows, axis=0)


def _embed(an8, w):
    return pl.pallas_call(
        _embed_body,
        out_shape=jax.ShapeDtypeStruct((8, _NT), jnp.float32),
        in_specs=[
            pl.BlockSpec(memory_space=pltpu.VMEM),
            pl.BlockSpec(memory_space=pltpu.SMEM),
        ],
        out_specs=pl.BlockSpec(memory_space=pltpu.VMEM),
    )(an8, w)


_BE = 512


def _edge_body(rv_ref, out_ref):
    r = rv_ref[0:1, :]
    vx = rv_ref[1:2, :]
    vy = rv_ref[2:3, :]
    vz = rv_ref[3:4, :]
    theta = (np.pi / _CUTOFF) * r
    s1 = jnp.sin(theta)
    c1 = jnp.cos(theta)
    sins = [s1, 2.0 * c1 * s1]
    for _ in range(4):
        sins.append(2.0 * c1 * sins[-1] - sins[-2])
    x = r * (1.0 / _CUTOFF)
    x2 = x * x
    x3 = x2 * x
    x6 = x3 * x3
    fc = 1.0 - 28.0 * x6 + 48.0 * x6 * x - 21.0 * x6 * x2
    fc = jnp.where(x < 1.0, fc, 0.0)
    pref = np.float32(np.sqrt(2.0 / _CUTOFF)) * fc / r
    rad = [pref * s for s in sins]
    one = jnp.ones(r.shape, jnp.float32)
    ang = [one, vx, vy, vz, vx * vx, vx * vy, vx * vz, vy * vy, vy * vz,
           vz * vz]
    zrow = jnp.zeros((1, r.shape[1]), jnp.float32)
    planes = []
    for r2 in range(32):
        cid, sid = divmod(r2, 16)
        pair = []
        for q in range(2):
            m = cid * _PM + 2 * sid + q
            pair.append(rad[m % 6] * ang[m // 6] if 2 * sid + q < _PM
                        else zrow)
        planes.append(jnp.concatenate(pair, axis=0)[None])
    out_ref[...] = jnp.concatenate(planes, axis=0)


def _edgefeat(rv):
    return pl.pallas_call(
        _edge_body,
        grid=(_EPAD // _BE,),
        out_shape=jax.ShapeDtypeStruct((32, 2, _EPAD), jnp.float32),
        in_specs=[pl.BlockSpec((8, _BE), lambda i: (0, i))],
        out_specs=pl.BlockSpec((32, 2, _BE), lambda i: (0, 0, i)),
    )(rv)


def _sc_body(pt_hbm, src_hbm, dst_hbm, emb_hbm, out_hbm,
             srcA, dstA, pvA, srcB, dstB, pvB, acc,
             embx, emby, embz, semA, semB):
    cid = lax.axis_index("c")
    sid = lax.axis_index("s")
    r2 = cid * 16 + sid

    pltpu.sync_copy(emb_hbm.at[0], embx)
    pltpu.sync_copy(emb_hbm.at[1], emby)
    pltpu.sync_copy(emb_hbm.at[2], embz)

    zero16 = jnp.zeros((16,), jnp.float32)
    lane = lax.iota(jnp.int32, 16)

    def _zrow(i, c):
        r16 = i * 16 + lane
        for cc in range(8):
            plsc.store_scatter(acc, [r16, jnp.full((16,), cc, jnp.int32)],
                               zero16)
        return c

    lax.fori_loop(0, _NT // 16, _zrow, 0)

    def _fire(b, sbuf, dbuf, pbuf, sem):
        eb = jnp.minimum(b, _NBLK - 1) * _BSZ
        pltpu.async_copy(src_hbm.at[pl.ds(eb, _BSZ)], sbuf, sem)
        pltpu.async_copy(dst_hbm.at[pl.ds(eb, _BSZ)], dbuf, sem)
        pltpu.async_copy(pt_hbm.at[r2, :, pl.ds(eb, _BSZ)], pbuf, sem)

    def _wait(sbuf, dbuf, pbuf, sem):
        pltpu.make_async_copy(src_hbm.at[pl.ds(0, _BSZ)], sbuf, sem).wait()
        pltpu.make_async_copy(dst_hbm.at[pl.ds(0, _BSZ)], dbuf, sem).wait()
        pltpu.make_async_copy(pt_hbm.at[0, :, pl.ds(0, _BSZ)], pbuf,
                              sem).wait()

    def _compute(sbuf, dbuf, pbuf):
        for g in range(_BSZ // 16):
            sl = pl.ds(g * 16, 16)
            s16 = sbuf[sl]
            d16 = dbuf[sl]
            ex = plsc.load_gather(embx, [s16])
            ey = plsc.load_gather(emby, [s16])
            ez = plsc.load_gather(embz, [s16])
            pm0 = pbuf[0, sl]
            pm1 = pbuf[1, sl]
            for q, pm in ((0, pm0), (1, pm1)):
                for a, ev in ((0, ex), (1, ey), (2, ez)):
                    col = jnp.full((16,), a * 2 + q, jnp.int32)
                    plsc.addupdate_scatter(acc, [d16, col], pm * ev)

    _fire(0, srcA, dstA, pvA, semA)

    def _body2(i, c):
        b0 = i * 2
        _fire(b0 + 1, srcB, dstB, pvB, semB)
        _wait(srcA, dstA, pvA, semA)
        _compute(srcA, dstA, pvA)
        _fire(b0 + 2, srcA, dstA, pvA, semA)
        _wait(srcB, dstB, pvB, semB)
        _compute(srcB, dstB, pvB)
        return c

    lax.fori_loop(0, _NBLK // 2, _body2, 0)
    _wait(srcA, dstA, pvA, semA)

    pltpu.sync_copy(acc, out_hbm.at[cid, sid])


_sc_call = functools.partial(
    pl.kernel,
    out_type=pltpu.HBM((2, 16, _NT, 8), jnp.float32),
    mesh=plsc.VectorSubcoreMesh(core_axis_name="c", subcore_axis_name="s",
                                num_cores=2, num_subcores=16),
    compiler_params=pltpu.CompilerParams(needs_layout_passes=False,
                                         use_tc_tiling_on_sc=False),
    scratch_types=[
        pltpu.VMEM((_BSZ,), jnp.int32),
        pltpu.VMEM((_BSZ,), jnp.int32),
        pltpu.VMEM((2, _BSZ), jnp.float32),
        pltpu.VMEM((_BSZ,), jnp.int32),
        pltpu.VMEM((_BSZ,), jnp.int32),
        pltpu.VMEM((2, _BSZ), jnp.float32),
        pltpu.VMEM((_NT, 8), jnp.float32),
        pltpu.VMEM((_NT,), jnp.float32),
        pltpu.VMEM((_NT,), jnp.float32),
        pltpu.VMEM((_NT,), jnp.float32),
        pltpu.SemaphoreType.DMA,
        pltpu.SemaphoreType.DMA,
    ],
)(_sc_body)


_BN = 256


def _combine_body(*refs):
    planes = refs[:32]
    emb_ref, r1_ref, r2_ref, rb_ref, a_ref, b_ref = refs[32:]
    s = jnp.concatenate([p[0, 0] for p in planes], axis=1)
    f = lax.dot_general(emb_ref[...], r2_ref[...], (((0,), (0,)), ((), ())),
                        preferred_element_type=jnp.float32)
    a = jnp.dot(s, r1_ref[...], preferred_element_type=jnp.float32) * f
    a_ref[...] = a
    aa = jnp.concatenate([a, a * a], axis=1)
    b_ref[...] = jnp.dot(aa, rb_ref[...], preferred_element_type=jnp.float32)


def _combine(s2, emb_cols, r1, r2, rb):
    plane_specs = [
        pl.BlockSpec((1, 1, _BN, 8), lambda i, c=c, s=s: (c, s, i, 0))
        for c in range(2) for s in range(16)
    ]
    return pl.pallas_call(
        _combine_body,
        grid=(_NT // _BN,),
        out_shape=(
            jax.ShapeDtypeStruct((_N, 540), jnp.float32),
            jax.ShapeDtypeStruct((_N, 162), jnp.float32),
        ),
        in_specs=plane_specs + [
            pl.BlockSpec((8, _BN), lambda i: (0, i)),
            pl.BlockSpec((256, 540), lambda i: (0, 0)),
            pl.BlockSpec((8, 540), lambda i: (0, 0)),
            pl.BlockSpec((1080, 162), lambda i: (0, 0)),
        ],
        out_specs=(
            pl.BlockSpec((_BN, 540), lambda i: (i, 0)),
            pl.BlockSpec((_BN, 162), lambda i: (i, 0)),
        ),
    )(*([s2] * 32), emb_cols, r1, r2, rb)


def kernel(positions, atomic_numbers, edge_index, edge_lengths, edge_vectors,
           W):
    src = edge_index[0].astype(jnp.int32)
    dst = edge_index[1].astype(jnp.int32)
    pad_e = _EPAD - _E
    src_p = jnp.concatenate([src, jnp.zeros((pad_e,), jnp.int32)])
    dst_p = jnp.concatenate([dst, jnp.full((pad_e,), _N, jnp.int32)])

    rv = jnp.concatenate([edge_lengths.reshape(1, _E),
                          edge_vectors.T.astype(jnp.float32),
                          jnp.zeros((4, _E), jnp.float32)], axis=0)
    rv = jnp.pad(rv, ((0, 0), (0, pad_e)), constant_values=1.0)

    an8 = jnp.pad(atomic_numbers.astype(jnp.int32).reshape(1, _N),
                  ((0, 7), (0, _NT - _N)))

    emb_cols = _embed(an8, W.astype(jnp.float32))
    pt = _edgefeat(rv)
    s2 = _sc_call(pt, src_p, dst_p, emb_cols)
    node_a, node_b = _combine(s2, emb_cols,
                              jnp.asarray(_R1), jnp.asarray(_R2),
                              jnp.asarray(_RB))
    return node_a.reshape(_N, 6, 10, 9), node_b.reshape(_N, 6, 3, 9)
